# Initial kernel scaffold; baseline (speedup 1.0000x reference)
import jax
import jax.numpy as jnp
from jax.experimental import pallas as pl


def kernel(x, temperature, embedding, log_var_q):
    raise NotImplementedError("write your pallas kernel here")


# speedup vs baseline: 3.4085x; 3.4085x over previous
import jax
import jax.numpy as jnp
from jax.experimental import pallas as pl
from jax.experimental.pallas import tpu as pltpu

_N_EMB = 512
_EMB_DIM = 64


def _body(params_ref, xt_ref, u_ref, embt_ref, emb_ref,
          qt_ref, loss_ref, perp_ref,
          hist_ref, sse_ref, ent_ref):
    i = pl.program_id(0)
    nb = pl.num_programs(0)
    precision = params_ref[0, 0]
    inv_temp = params_ref[0, 1]

    xb = xt_ref[0]
    embt = embt_ref[...]
    e2 = jnp.sum(embt * embt, axis=0, keepdims=True)
    x2 = jnp.sum(xb * xb, axis=1, keepdims=True)
    xy = jnp.dot(xb, embt, preferred_element_type=jnp.float32,
                 precision=jax.lax.Precision.HIGHEST)
    dist = (0.5 * precision) * (x2 - 2.0 * xy + e2)
    logits = -dist

    mx = jnp.max(logits, axis=1, keepdims=True)
    ex = jnp.exp(logits - mx)
    s = jnp.sum(ex, axis=1, keepdims=True)
    lse = mx + jnp.log(s)
    plogp = (ex / s) * (logits - lse)

    g = -jnp.log(-jnp.log(u_ref[0]))
    gl = (logits + g) * inv_temp
    gmx = jnp.max(gl, axis=1, keepdims=True)
    ge = jnp.exp(gl - gmx)
    gs = jnp.sum(ge, axis=1, keepdims=True)
    enc = ge / gs
    q = jnp.dot(enc, emb_ref[...], preferred_element_type=jnp.float32,
                precision=jax.lax.Precision.HIGHEST)
    qt_ref[0] = q

    dmin = jnp.min(dist, axis=1, keepdims=True)
    lane = jax.lax.broadcasted_iota(jnp.int32, dist.shape, 1)
    cand = jnp.where(dist == dmin, lane, _N_EMB)
    amin = jnp.min(cand, axis=1, keepdims=True)
    onehot = (lane == amin).astype(jnp.float32)

    hpart = jnp.sum(onehot, axis=0, keepdims=True)
    entp = jnp.sum(plogp, axis=0, keepdims=True)
    ssep = jnp.sum((xb - q) ** 2, axis=0, keepdims=True)

    @pl.when(i == 0)
    def _init():
        hist_ref[...] = jnp.zeros_like(hist_ref)
        ent_ref[...] = jnp.zeros_like(ent_ref)
        sse_ref[...] = jnp.zeros_like(sse_ref)

    hist_ref[...] += hpart
    ent_ref[...] += entp
    sse_ref[...] += ssep

    @pl.when(i == nb - 1)
    def _finish():
        n_tok = nb * xb.shape[0]
        avg = hist_ref[...] / jnp.float32(n_tok)
        perp = jnp.exp(-jnp.sum(avg * jnp.log(avg + 1e-10)))
        sse = jnp.sum(sse_ref[...])
        ent = jnp.sum(ent_ref[...])
        loss_ref[0, 0] = (0.5 * precision * sse + ent) / jnp.float32(nb)
        perp_ref[0, 0] = perp


def kernel(x, temperature, embedding, log_var_q):
    B, D, T = x.shape
    M, Dd = embedding.shape
    xt = jnp.transpose(x, (0, 2, 1))
    embt = embedding.T
    eps = jnp.finfo(jnp.float32).eps
    u = jax.random.uniform(jax.random.key(42), (B * T, M), jnp.float32,
                           minval=eps, maxval=1.0 - eps).reshape(B, T, M)
    precision = jnp.exp(-log_var_q).astype(jnp.float32)
    inv_temp = (1.0 / temperature).astype(jnp.float32)
    params = jnp.stack([precision, inv_temp]).reshape(1, 2)

    qt, loss, perp = pl.pallas_call(
        _body,
        grid=(B,),
        in_specs=[
            pl.BlockSpec((1, 2), lambda i: (0, 0), memory_space=pltpu.SMEM),
            pl.BlockSpec((1, T, D), lambda i: (i, 0, 0)),
            pl.BlockSpec((1, T, M), lambda i: (i, 0, 0)),
            pl.BlockSpec((D, M), lambda i: (0, 0)),
            pl.BlockSpec((M, D), lambda i: (0, 0)),
        ],
        out_specs=[
            pl.BlockSpec((1, T, D), lambda i: (i, 0, 0)),
            pl.BlockSpec((1, 1), lambda i: (0, 0), memory_space=pltpu.SMEM),
            pl.BlockSpec((1, 1), lambda i: (0, 0), memory_space=pltpu.SMEM),
        ],
        out_shape=[
            jax.ShapeDtypeStruct((B, T, D), jnp.float32),
            jax.ShapeDtypeStruct((1, 1), jnp.float32),
            jax.ShapeDtypeStruct((1, 1), jnp.float32),
        ],
        scratch_shapes=[
            pltpu.VMEM((1, M), jnp.float32),
            pltpu.VMEM((1, D), jnp.float32),
            pltpu.VMEM((1, M), jnp.float32),
        ],
        compiler_params=pltpu.CompilerParams(
            dimension_semantics=("arbitrary",)),
    )(params, xt, u, embt, embedding)

    quantized = jnp.transpose(qt, (0, 2, 1))
    return quantized, loss[0, 0], perp[0, 0]


# speedup vs baseline: 4.3821x; 1.2856x over previous
import jax
import jax.numpy as jnp
from jax.experimental import pallas as pl
from jax.experimental.pallas import tpu as pltpu

_N_EMB = 512


def _body(params_ref, x_ref, g_ref, emb_ref,
          q_ref, loss_ref, perp_ref,
          hist_ref, sse_ref, ent_ref):
    i = pl.program_id(0)
    nb = pl.num_programs(0)
    precision = params_ref[0, 0]
    inv_temp = params_ref[0, 1]

    xb = x_ref[0]
    emb = emb_ref[...]
    e2 = jnp.sum(emb * emb, axis=1, keepdims=True)
    x2 = jnp.sum(xb * xb, axis=0, keepdims=True)
    xy = jnp.dot(emb, xb, preferred_element_type=jnp.float32,
                 precision=jax.lax.Precision.HIGHEST)
    dist = (0.5 * precision) * (e2 - 2.0 * xy + x2)
    logits = -dist

    mx = jnp.max(logits, axis=0, keepdims=True)
    ex = jnp.exp(logits - mx)
    s = jnp.sum(ex, axis=0, keepdims=True)
    lse = mx + jnp.log(s)
    plogp = (ex * (1.0 / s)) * (logits - lse)

    gl = (logits + g_ref[0]) * inv_temp
    gmx = jnp.max(gl, axis=0, keepdims=True)
    ge = jnp.exp(gl - gmx)
    gs = jnp.sum(ge, axis=0, keepdims=True)
    enc = ge * (1.0 / gs)
    q = jax.lax.dot_general(emb, enc, (((0,), (0,)), ((), ())),
                            preferred_element_type=jnp.float32)
    q_ref[0] = q

    dmin = jnp.min(dist, axis=0, keepdims=True)
    row = jax.lax.broadcasted_iota(jnp.int32, dist.shape, 0)
    cand = jnp.where(dist == dmin, row, _N_EMB)
    amin = jnp.min(cand, axis=0, keepdims=True)
    onehot = (row == amin).astype(jnp.float32)

    hpart = jnp.sum(onehot, axis=1, keepdims=True)
    entp = jnp.sum(plogp, axis=0, keepdims=True)
    ssep = jnp.sum((xb - q) ** 2, axis=0, keepdims=True)

    @pl.when(i == 0)
    def _init():
        hist_ref[...] = jnp.zeros_like(hist_ref)
        ent_ref[...] = jnp.zeros_like(ent_ref)
        sse_ref[...] = jnp.zeros_like(sse_ref)

    hist_ref[...] += hpart
    ent_ref[...] += entp
    sse_ref[...] += ssep

    @pl.when(i == nb - 1)
    def _finish():
        n_tok = nb * xb.shape[1]
        avg = hist_ref[...] / jnp.float32(n_tok)
        perp = jnp.exp(-jnp.sum(avg * jnp.log(avg + 1e-10)))
        sse = jnp.sum(sse_ref[...])
        ent = jnp.sum(ent_ref[...])
        loss_ref[0, 0] = (0.5 * precision * sse + ent) / jnp.float32(nb)
        perp_ref[0, 0] = perp


def kernel(x, temperature, embedding, log_var_q):
    B, D, T = x.shape
    M, _ = embedding.shape
    eps = jnp.finfo(jnp.float32).eps
    u = jax.random.uniform(jax.random.key(42), (B * T, M), jnp.float32,
                           minval=eps, maxval=1.0 - eps)
    g = jnp.transpose((-jnp.log(-jnp.log(u))).reshape(B, T, M), (0, 2, 1))

    precision = jnp.exp(-log_var_q).astype(jnp.float32)
    inv_temp = (1.0 / temperature).astype(jnp.float32)
    params = jnp.stack([precision, inv_temp]).reshape(1, 2)

    q, loss, perp = pl.pallas_call(
        _body,
        grid=(B,),
        in_specs=[
            pl.BlockSpec((1, 2), lambda i: (0, 0), memory_space=pltpu.SMEM),
            pl.BlockSpec((1, D, T), lambda i: (i, 0, 0)),
            pl.BlockSpec((1, M, T), lambda i: (i, 0, 0)),
            pl.BlockSpec((M, D), lambda i: (0, 0)),
        ],
        out_specs=[
            pl.BlockSpec((1, D, T), lambda i: (i, 0, 0)),
            pl.BlockSpec((1, 1), lambda i: (0, 0), memory_space=pltpu.SMEM),
            pl.BlockSpec((1, 1), lambda i: (0, 0), memory_space=pltpu.SMEM),
        ],
        out_shape=[
            jax.ShapeDtypeStruct((B, D, T), jnp.float32),
            jax.ShapeDtypeStruct((1, 1), jnp.float32),
            jax.ShapeDtypeStruct((1, 1), jnp.float32),
        ],
        scratch_shapes=[
            pltpu.VMEM((M, 1), jnp.float32),
            pltpu.VMEM((1, T), jnp.float32),
            pltpu.VMEM((1, T), jnp.float32),
        ],
        compiler_params=pltpu.CompilerParams(
            dimension_semantics=("arbitrary",)),
    )(params, x, g, embedding)

    return q, loss[0, 0], perp[0, 0]


# speedup vs baseline: 4.6942x; 1.0712x over previous
import jax
import jax.numpy as jnp
from jax.experimental import pallas as pl
from jax.experimental.pallas import tpu as pltpu

_LOG2E = 1.4426950408889634


def _body(params_ref, x_ref, g_ref, emb_ref,
          q_ref, loss_ref, perp_ref,
          hist_ref, sse_ref, ent_ref):
    i = pl.program_id(0)
    nb = pl.num_programs(0)
    c = 0.5 * params_ref[0, 0]
    inv_temp = params_ref[0, 1]

    xb = x_ref[0]
    emb = emb_ref[...]
    emb2c = (c + c) * emb
    ce2 = 0.5 * jnp.sum(emb2c * emb, axis=1, keepdims=True)
    xy2 = jnp.dot(emb2c, xb, preferred_element_type=jnp.float32,
                  precision=jax.lax.Precision.HIGHEST)
    ncr = xy2 - ce2

    nmax = jnp.max(ncr, axis=0, keepdims=True)
    t = ncr - nmax
    ex = jnp.exp2(t * _LOG2E)
    s = jnp.sum(ex, axis=0, keepdims=True)
    sxt = jnp.sum(ex * t, axis=0, keepdims=True)
    entp = sxt * (1.0 / s) - jnp.log(s)

    h = ncr + g_ref[0]
    hmax = jnp.max(h, axis=0, keepdims=True)
    ge = jnp.exp2((h - hmax) * (inv_temp * _LOG2E))
    gs = jnp.sum(ge, axis=0, keepdims=True)
    q = jax.lax.dot_general(emb, ge, (((0,), (0,)), ((), ())),
                            preferred_element_type=jnp.float32)
    q = q * (1.0 / gs)
    q_ref[0] = q

    hpart = jnp.sum((ncr == nmax).astype(jnp.float32), axis=1,
                    keepdims=True)
    ssep = jnp.sum((xb - q) ** 2, axis=0, keepdims=True)

    @pl.when(i == 0)
    def _init():
        hist_ref[...] = jnp.zeros_like(hist_ref)
        ent_ref[...] = jnp.zeros_like(ent_ref)
        sse_ref[...] = jnp.zeros_like(sse_ref)

    hist_ref[...] += hpart
    ent_ref[...] += entp
    sse_ref[...] += ssep

    @pl.when(i == nb - 1)
    def _finish():
        n_tok = nb * xb.shape[1]
        avg = hist_ref[...] / jnp.float32(n_tok)
        perp = jnp.exp(-jnp.sum(avg * jnp.log(avg + 1e-10)))
        sse = jnp.sum(sse_ref[...])
        ent = jnp.sum(ent_ref[...])
        loss_ref[0, 0] = (c * sse + ent) / jnp.float32(nb)
        perp_ref[0, 0] = perp


def kernel(x, temperature, embedding, log_var_q):
    B, D, T = x.shape
    M, _ = embedding.shape
    eps = jnp.finfo(jnp.float32).eps
    u = jax.random.uniform(jax.random.key(42), (B * T, M), jnp.float32,
                           minval=eps, maxval=1.0 - eps)
    g = jnp.transpose((-jnp.log(-jnp.log(u))).reshape(B, T, M), (0, 2, 1))

    precision = jnp.exp(-log_var_q).astype(jnp.float32)
    inv_temp = (1.0 / temperature).astype(jnp.float32)
    params = jnp.stack([precision, inv_temp]).reshape(1, 2)

    q, loss, perp = pl.pallas_call(
        _body,
        grid=(B,),
        in_specs=[
            pl.BlockSpec((1, 2), lambda i: (0, 0), memory_space=pltpu.SMEM),
            pl.BlockSpec((1, D, T), lambda i: (i, 0, 0)),
            pl.BlockSpec((1, M, T), lambda i: (i, 0, 0)),
            pl.BlockSpec((M, D), lambda i: (0, 0)),
        ],
        out_specs=[
            pl.BlockSpec((1, D, T), lambda i: (i, 0, 0)),
            pl.BlockSpec((1, 1), lambda i: (0, 0), memory_space=pltpu.SMEM),
            pl.BlockSpec((1, 1), lambda i: (0, 0), memory_space=pltpu.SMEM),
        ],
        out_shape=[
            jax.ShapeDtypeStruct((B, D, T), jnp.float32),
            jax.ShapeDtypeStruct((1, 1), jnp.float32),
            jax.ShapeDtypeStruct((1, 1), jnp.float32),
        ],
        scratch_shapes=[
            pltpu.VMEM((M, 1), jnp.float32),
            pltpu.VMEM((1, T), jnp.float32),
            pltpu.VMEM((1, T), jnp.float32),
        ],
        compiler_params=pltpu.CompilerParams(
            dimension_semantics=("arbitrary",)),
    )(params, x, g, embedding)

    return q, loss[0, 0], perp[0, 0]
